# baseline (speedup 1.0000x reference)
import jax
import jax.numpy as jnp
from jax import lax
from jax.experimental import pallas as pl
from jax.experimental.pallas import tpu as pltpu, tpu_sc as plsc


_NC, _NS, _L = 2, 16, 16
_NW = _NC * _NS
_NR = 4


def _sc_body(x_hbm, pos_hbm, out_hbm,
             pb0, pb1, xb0, xb1, xb2, xb3,
             sp0, sp1, si0, si1, si2, si3, so0, so1, so2, so3):
    B, S, D = 4, 2048, 4096
    ch = _NR * D
    c = lax.axis_index("c")
    s = lax.axis_index("s")
    wid = s * _NC + c
    rows_per_w = S // _NW
    base_row = wid * rows_per_w
    nj = rows_per_w // _NR
    nsteps = nj * B

    xbs = [xb0, xb1, xb2, xb3]
    sis = [si0, si1, si2, si3]
    sos = [so0, so1, so2, so3]
    pbs = [pb0, pb1]
    sps = [sp0, sp1]

    def x_off(k):
        j = k // B
        b = k - j * B
        return (b * S + base_row + j * _NR) * D

    def p_off(j):
        return (base_row + j * _NR) * D

    def add_loop(xb, pb):
        @plsc.parallel_loop(0, ch, _L, unroll=8)
        def _(o):
            oa = pl.multiple_of(o, _L)
            plsc.addupdate(xb.at[pl.ds(oa, _L)], pb[pl.ds(oa, _L)])

    pltpu.async_copy(pos_hbm.at[pl.ds(p_off(0), ch)], pb0, sp0)
    pltpu.async_copy(x_hbm.at[pl.ds(x_off(0), ch)], xb0, si0)
    pltpu.async_copy(x_hbm.at[pl.ds(x_off(1), ch)], xb1, si1)

    def gg_loop(gg, _):
        for dj in range(2):
            g = 2 * gg + dj
            pltpu.make_async_copy(pos_hbm.at[pl.ds(p_off(g), ch)],
                                  pbs[dj], sps[dj]).wait()
            @pl.when(g + 1 < nj)
            def _():
                pltpu.async_copy(pos_hbm.at[pl.ds(p_off(g + 1), ch)],
                                 pbs[1 - dj], sps[1 - dj])
            for d in range(4):
                k = 4 * g + d
                dn = (d + 2) % 4
                @pl.when(k >= 2)
                def _():
                    pltpu.make_async_copy(
                        xbs[dn], out_hbm.at[pl.ds(x_off(k - 2), ch)],
                        sos[dn]).wait()
                @pl.when(k + 2 < nsteps)
                def _():
                    pltpu.async_copy(x_hbm.at[pl.ds(x_off(k + 2), ch)],
                                     xbs[dn], sis[dn])
                pltpu.make_async_copy(x_hbm.at[pl.ds(x_off(k), ch)],
                                      xbs[d], sis[d]).wait()
                add_loop(xbs[d], pbs[dj])
                pltpu.async_copy(xbs[d], out_hbm.at[pl.ds(x_off(k), ch)],
                                 sos[d])
        return 0

    lax.fori_loop(0, nj // 2, gg_loop, 0)

    pltpu.make_async_copy(xb2, out_hbm.at[pl.ds(x_off(nsteps - 2), ch)],
                          so2).wait()
    pltpu.make_async_copy(xb3, out_hbm.at[pl.ds(x_off(nsteps - 1), ch)],
                          so3).wait()


def kernel(x, pos_table):
    B, S, D = x.shape
    ch = _NR * D
    mesh = plsc.VectorSubcoreMesh(core_axis_name="c", subcore_axis_name="s")
    out_flat = pl.kernel(
        _sc_body,
        out_type=jax.ShapeDtypeStruct((B * S * D,), jnp.float32),
        mesh=mesh,
        scratch_types=(
            [pltpu.VMEM((ch,), jnp.float32) for _ in range(6)]
            + [pltpu.SemaphoreType.DMA for _ in range(10)]
        ),
    )(x.reshape(-1), pos_table.reshape(-1))
    return out_flat.reshape(B, S, D)


# speedup vs baseline: 1.0029x; 1.0029x over previous
import jax
import jax.numpy as jnp
from jax import lax
from jax.experimental import pallas as pl
from jax.experimental.pallas import tpu as pltpu, tpu_sc as plsc


_NC, _NS, _L = 2, 16, 16
_NW = _NC * _NS
_NR = 4


def _sc_body(x_hbm, pos_hbm, out_hbm,
             pb0, pb1, xb0, xb1, xb2, xb3,
             sp0, sp1, si0, si1, si2, si3, so0, so1, so2, so3):
    B, S, D = 4, 2048, 4096
    ch = _NR * D
    c = lax.axis_index("c")
    s = lax.axis_index("s")
    wid = s * _NC + c
    rows_per_w = S // _NW
    base_row = wid * rows_per_w
    nj = rows_per_w // _NR
    nsteps = nj * B

    xbs = [xb0, xb1, xb2, xb3]
    sis = [si0, si1, si2, si3]
    sos = [so0, so1, so2, so3]
    pbs = [pb0, pb1]
    sps = [sp0, sp1]

    def x_off(k):
        j = k // B
        b = k - j * B
        return (b * S + base_row + j * _NR) * D

    def p_off(j):
        return (base_row + j * _NR) * D

    def add_loop(xb, pb):
        @plsc.parallel_loop(0, ch, _L, unroll=8)
        def _(o):
            oa = pl.multiple_of(o, _L)
            plsc.addupdate(xb.at[pl.ds(oa, _L)], pb[pl.ds(oa, _L)])

    pltpu.async_copy(pos_hbm.at[pl.ds(p_off(0), ch)], pb0, sp0)
    pltpu.async_copy(x_hbm.at[pl.ds(x_off(0), ch)], xb0, si0)
    pltpu.async_copy(x_hbm.at[pl.ds(x_off(1), ch)], xb1, si1)

    def gg_loop(gg, _):
        for dj in range(2):
            g = 2 * gg + dj
            pltpu.make_async_copy(pos_hbm.at[pl.ds(p_off(g), ch)],
                                  pbs[dj], sps[dj]).wait()
            @pl.when(g + 1 < nj)
            def _():
                pltpu.async_copy(pos_hbm.at[pl.ds(p_off(g + 1), ch)],
                                 pbs[1 - dj], sps[1 - dj])
            for d in range(4):
                k = 4 * g + d
                dn = (d + 2) % 4
                @pl.when(k >= 2)
                def _():
                    pltpu.make_async_copy(
                        xbs[dn], out_hbm.at[pl.ds(x_off(k - 2), ch)],
                        sos[dn]).wait()
                @pl.when(k + 2 < nsteps)
                def _():
                    pltpu.async_copy(x_hbm.at[pl.ds(x_off(k + 2), ch)],
                                     xbs[dn], sis[dn])
                pltpu.make_async_copy(x_hbm.at[pl.ds(x_off(k), ch)],
                                      xbs[d], sis[d]).wait()
                pltpu.async_copy(xbs[d], out_hbm.at[pl.ds(x_off(k), ch)],
                                 sos[d])
        return 0

    lax.fori_loop(0, nj // 2, gg_loop, 0)

    pltpu.make_async_copy(xb2, out_hbm.at[pl.ds(x_off(nsteps - 2), ch)],
                          so2).wait()
    pltpu.make_async_copy(xb3, out_hbm.at[pl.ds(x_off(nsteps - 1), ch)],
                          so3).wait()


def kernel(x, pos_table):
    B, S, D = x.shape
    ch = _NR * D
    mesh = plsc.VectorSubcoreMesh(core_axis_name="c", subcore_axis_name="s")
    out_flat = pl.kernel(
        _sc_body,
        out_type=jax.ShapeDtypeStruct((B * S * D,), jnp.float32),
        mesh=mesh,
        scratch_types=(
            [pltpu.VMEM((ch,), jnp.float32) for _ in range(6)]
            + [pltpu.SemaphoreType.DMA for _ in range(10)]
        ),
    )(x.reshape(-1), pos_table.reshape(-1))
    return out_flat.reshape(B, S, D)


# speedup vs baseline: 4.1319x; 4.1198x over previous
import jax
import jax.numpy as jnp
from jax.experimental import pallas as pl


_BS = 512


def _add_kernel(x_ref, pos_ref, out_ref):
    out_ref[...] = x_ref[...] + pos_ref[...][None]


def kernel(x, pos_table):
    B, S, D = x.shape
    bs = _BS if S % _BS == 0 else S
    grid = (S // bs, B)
    return pl.pallas_call(
        _add_kernel,
        grid=grid,
        in_specs=[
            pl.BlockSpec((1, bs, D), lambda s, b: (b, s, 0)),
            pl.BlockSpec((bs, D), lambda s, b: (s, 0)),
        ],
        out_specs=pl.BlockSpec((1, bs, D), lambda s, b: (b, s, 0)),
        out_shape=jax.ShapeDtypeStruct((B, S, D), x.dtype),
    )(x, pos_table)
